# Initial kernel scaffold; baseline (speedup 1.0000x reference)
import jax
import jax.numpy as jnp
from jax.experimental import pallas as pl


def kernel(x, table):
    raise NotImplementedError("write your pallas kernel here")


# speedup vs baseline: 2.6404x; 2.6404x over previous
import jax
import jax.numpy as jnp
from jax.experimental import pallas as pl


def _bcast_body(t_ref, o_ref):
    o_ref[...] = jnp.broadcast_to(t_ref[...][None], o_ref.shape)


def kernel(x, table):
    batch, length = x.shape
    feat = table.shape[1]
    block_l = 256
    grid = (pl.cdiv(length, block_l),)
    return pl.pallas_call(
        _bcast_body,
        grid=grid,
        in_specs=[pl.BlockSpec((block_l, feat), lambda i: (i, 0))],
        out_specs=pl.BlockSpec((batch, block_l, feat), lambda i: (0, i, 0)),
        out_shape=jax.ShapeDtypeStruct((batch, length, feat), table.dtype),
    )(table)


# speedup vs baseline: 2.7156x; 1.0285x over previous
import jax
import jax.numpy as jnp
from jax.experimental import pallas as pl


def _bcast_body(t_ref, o_ref):
    o_ref[...] = jnp.broadcast_to(t_ref[...][None], o_ref.shape)


def kernel(x, table):
    batch, length = x.shape
    feat = table.shape[1]
    block_l = 512
    grid = (pl.cdiv(length, block_l),)
    return pl.pallas_call(
        _bcast_body,
        grid=grid,
        in_specs=[pl.BlockSpec((block_l, feat), lambda i: (i, 0))],
        out_specs=pl.BlockSpec((batch, block_l, feat), lambda i: (0, i, 0)),
        out_shape=jax.ShapeDtypeStruct((batch, length, feat), table.dtype),
    )(table)


# speedup vs baseline: 2.8360x; 1.0443x over previous
import jax
import jax.numpy as jnp
from jax.experimental import pallas as pl
from jax.experimental.pallas import tpu as pltpu

_N_CHUNKS = 8


def _copy_body(t_hbm, o_hbm, vmem, sems_in, sem_out):
    length = vmem.shape[0]
    batch = o_hbm.shape[0]
    ch = length // _N_CHUNKS
    for c in range(_N_CHUNKS):
        pltpu.make_async_copy(
            t_hbm.at[pl.ds(c * ch, ch)], vmem.at[pl.ds(c * ch, ch)], sems_in.at[c]
        ).start()
    for c in range(_N_CHUNKS):
        pltpu.make_async_copy(
            t_hbm.at[pl.ds(c * ch, ch)], vmem.at[pl.ds(c * ch, ch)], sems_in.at[c]
        ).wait()
        for b in range(batch):
            pltpu.make_async_copy(
                vmem.at[pl.ds(c * ch, ch)], o_hbm.at[b, pl.ds(c * ch, ch)], sem_out
            ).start()
    for c in range(_N_CHUNKS):
        for b in range(batch):
            pltpu.make_async_copy(
                vmem.at[pl.ds(c * ch, ch)], o_hbm.at[b, pl.ds(c * ch, ch)], sem_out
            ).wait()


def kernel(x, table):
    batch, length = x.shape
    feat = table.shape[1]
    return pl.pallas_call(
        _copy_body,
        in_specs=[pl.BlockSpec(memory_space=pl.ANY)],
        out_specs=pl.BlockSpec(memory_space=pl.ANY),
        out_shape=jax.ShapeDtypeStruct((batch, length, feat), table.dtype),
        scratch_shapes=[
            pltpu.VMEM((length, feat), table.dtype),
            pltpu.SemaphoreType.DMA((_N_CHUNKS,)),
            pltpu.SemaphoreType.DMA,
        ],
    )(table)
